# baseline (speedup 1.0000x reference)
import jax
import jax.numpy as jnp
from jax.experimental import pallas as pl
from jax.experimental.pallas import tpu as pltpu

_N_BLOCK = 10000


def _dot_t(a, w):
    return jax.lax.dot_general(
        a, w, (((1,), (1,)), ((), ())), preferred_element_type=jnp.float32
    )


def _gelu(x):
    return 0.5 * x * (1.0 + jax.lax.erf(x * 0.7071067811865476))


def _ln(h, g, b):
    mu = jnp.mean(h, axis=-1, keepdims=True)
    d = h - mu
    var = jnp.mean(d * d, axis=-1, keepdims=True)
    return d * jax.lax.rsqrt(var + 1e-5) * g + b


def _fused_mlp_kernel(
    x_ref,
    w0_ref, b0_ref, g0_ref, be0_ref,
    w1_ref, b1_ref, g1_ref, be1_ref,
    wo1_ref, bo1_ref,
    wo2_ref, bo2_ref,
    o_ref,
):
    x = x_ref[...]
    h = _gelu(_dot_t(x, w0_ref[...]) + b0_ref[...])
    h = _ln(h, g0_ref[...], be0_ref[...])
    h = _gelu(_dot_t(h, w1_ref[...]) + b1_ref[...])
    h = _ln(h, g1_ref[...], be1_ref[...])
    h = h + h
    o = _gelu(_dot_t(h, wo1_ref[...]) + bo1_ref[...])
    o_ref[...] = _dot_t(o, wo2_ref[...]) + bo2_ref[...]


def kernel(x, edge_index, W0, b0, g0, be0, W1, b1, g1, be1, Wo1, bo1, Wo2, bo2):
    del edge_index
    n, d = x.shape
    o = Wo2.shape[0]
    row2 = lambda v: v.reshape(1, -1)

    grid = (pl.cdiv(n, _N_BLOCK),)
    full = lambda a: pl.BlockSpec(a.shape, lambda i: (0,) * a.ndim)

    args = (
        x,
        W0, row2(b0), row2(g0), row2(be0),
        W1, row2(b1), row2(g1), row2(be1),
        Wo1, row2(bo1),
        Wo2, row2(bo2),
    )
    in_specs = [pl.BlockSpec((_N_BLOCK, d), lambda i: (i, 0))] + [
        full(a) for a in args[1:]
    ]
    return pl.pallas_call(
        _fused_mlp_kernel,
        grid=grid,
        in_specs=in_specs,
        out_specs=pl.BlockSpec((_N_BLOCK, o), lambda i: (i, 0)),
        out_shape=jax.ShapeDtypeStruct((n, o), jnp.float32),
        compiler_params=pltpu.CompilerParams(
            dimension_semantics=("parallel",),
        ),
    )(*args)


# speedup vs baseline: 1.1742x; 1.1742x over previous
import jax
import jax.numpy as jnp
from jax.experimental import pallas as pl
from jax.experimental.pallas import tpu as pltpu

_N_BLOCK = 5000


def _dot_t(a, w):
    return jax.lax.dot_general(
        a, w, (((1,), (1,)), ((), ())), preferred_element_type=jnp.float32
    )


def _gelu(x):
    return 0.5 * x * (1.0 + jax.lax.erf(x * 0.7071067811865476))


def _ln_noaffine(h):
    mu = jnp.mean(h, axis=-1, keepdims=True)
    d = h - mu
    var = jnp.mean(d * d, axis=-1, keepdims=True)
    return d * jax.lax.rsqrt(var + 1e-5)


def _fused_mlp_kernel(
    x_ref,
    w0_ref, b0_ref, g0_ref, be0_ref,
    w1_ref, b1_ref, g1_ref, be1_ref,
    wo1_ref, bo1_ref,
    wo2_ref, bo2_ref,
    o_ref,
):
    w1f = w1_ref[...] * g0_ref[...]
    b1f = b1_ref[...] + _dot_t(be0_ref[...], w1_ref[...])
    wo1f = wo1_ref[...] * (2.0 * g1_ref[...])
    bo1f = bo1_ref[...] + 2.0 * _dot_t(be1_ref[...], wo1_ref[...])

    x = x_ref[...]
    h = _gelu(_dot_t(x, w0_ref[...]) + b0_ref[...])
    h = _ln_noaffine(h)
    h = _gelu(_dot_t(h, w1f) + b1f)
    h = _ln_noaffine(h)
    o = _gelu(_dot_t(h, wo1f) + bo1f)
    o_ref[...] = _dot_t(o, wo2_ref[...]) + bo2_ref[...]


def kernel(x, edge_index, W0, b0, g0, be0, W1, b1, g1, be1, Wo1, bo1, Wo2, bo2):
    del edge_index
    n, d = x.shape
    o = Wo2.shape[0]
    row2 = lambda v: v.reshape(1, -1)

    grid = (pl.cdiv(n, _N_BLOCK),)
    full = lambda a: pl.BlockSpec(a.shape, lambda i: (0,) * a.ndim)

    args = (
        x,
        W0, row2(b0), row2(g0), row2(be0),
        W1, row2(b1), row2(g1), row2(be1),
        Wo1, row2(bo1),
        Wo2, row2(bo2),
    )
    in_specs = [pl.BlockSpec((_N_BLOCK, d), lambda i: (i, 0))] + [
        full(a) for a in args[1:]
    ]
    return pl.pallas_call(
        _fused_mlp_kernel,
        grid=grid,
        in_specs=in_specs,
        out_specs=pl.BlockSpec((_N_BLOCK, o), lambda i: (i, 0)),
        out_shape=jax.ShapeDtypeStruct((n, o), jnp.float32),
        compiler_params=pltpu.CompilerParams(
            dimension_semantics=("parallel",),
        ),
    )(*args)
